# baseline (speedup 1.0000x reference)
import functools
import math

import jax
import jax.numpy as jnp
from jax import lax
from jax.experimental import pallas as pl
from jax.experimental.pallas import tpu as pltpu
from jax.experimental.pallas import tpu_sc as plsc

D = 64
EPS = 1e-6
SQRT_D = math.sqrt(D)

NC = 2
NS = 16
NW = NC * NS

CHUNK = 128


def _sc_gather(table, idx):
    n = idx.shape[0]
    per_w = n // NW
    n_groups = per_w // CHUNK
    mesh = plsc.VectorSubcoreMesh(core_axis_name="c", subcore_axis_name="s")

    @functools.partial(
        pl.kernel,
        out_type=jax.ShapeDtypeStruct((n, D), jnp.float32),
        mesh=mesh,
        scratch_types=[
            pltpu.VMEM((CHUNK,), jnp.int32),
            pltpu.VMEM((CHUNK, D), jnp.float32),
            pltpu.SemaphoreType.DMA,
        ],
        compiler_params=pltpu.CompilerParams(use_tc_tiling_on_sc=False),
    )
    def gather_kernel(table_hbm, idx_hbm, out_hbm, idx_v, rows_v, sem):
        wid = lax.axis_index("s") * NC + lax.axis_index("c")
        w_base = wid * per_w

        def body(g, carry):
            base = w_base + g * CHUNK
            pltpu.sync_copy(idx_hbm.at[pl.ds(base, CHUNK)], idx_v)
            pltpu.async_copy(table_hbm.at[idx_v], rows_v, sem).wait()
            pltpu.sync_copy(rows_v, out_hbm.at[pl.ds(base, CHUNK)])
            return carry

        lax.fori_loop(0, n_groups, body, 0)

    return gather_kernel(table, idx)


def _ln_body(emb_ref, pos_ref, gamma_ref, beta_ref, out_ref):
    h = emb_ref[...] * SQRT_D + pos_ref[...]
    mean = jnp.mean(h, axis=-1, keepdims=True)
    d = h - mean
    var = jnp.sum(d * d, axis=-1, keepdims=True) * (1.0 / (D - 1))
    out_ref[...] = gamma_ref[...] * (d / (jnp.sqrt(var) + EPS)) + beta_ref[...]


def _tc_layernorm(emb, pos, gamma, beta):
    b, s, d = emb.shape
    bb = 32
    grid = (b // bb,)
    return pl.pallas_call(
        _ln_body,
        grid=grid,
        in_specs=[
            pl.BlockSpec((bb, s, d), lambda i: (i, 0, 0)),
            pl.BlockSpec((1, s, d), lambda i: (0, 0, 0)),
            pl.BlockSpec((1, 1, d), lambda i: (0, 0, 0)),
            pl.BlockSpec((1, 1, d), lambda i: (0, 0, 0)),
        ],
        out_specs=pl.BlockSpec((bb, s, d), lambda i: (i, 0, 0)),
        out_shape=jax.ShapeDtypeStruct((b, s, d), jnp.float32),
    )(emb, pos, gamma, beta)


def kernel(x, table, gamma, beta, pos_embed):
    b, s = x.shape
    idx = x.reshape(-1).astype(jnp.int32)
    gathered = _sc_gather(table, idx)
    emb = gathered.reshape(b, s, D)
    pos = lax.slice(pos_embed, (0, 1, 0), (1, s + 1, D))
    return _tc_layernorm(emb, pos, gamma.reshape(1, 1, D), beta.reshape(1, 1, D))


# speedup vs baseline: 1.1433x; 1.1433x over previous
import functools
import math

import jax
import jax.numpy as jnp
from jax import lax
from jax.experimental import pallas as pl
from jax.experimental.pallas import tpu as pltpu
from jax.experimental.pallas import tpu_sc as plsc

D = 64
EPS = 1e-6
SQRT_D = math.sqrt(D)

NC = 2
NS = 16
NW = NC * NS

CHUNK = 128


BURST = 4


def _sc_gather(table, idx2):
    n_rows = idx2.shape[0]
    n = n_rows * CHUNK
    rows_per_w = n_rows // NW
    n_phases = rows_per_w // BURST
    mesh = plsc.VectorSubcoreMesh(core_axis_name="c", subcore_axis_name="s")

    @functools.partial(
        pl.kernel,
        out_type=jax.ShapeDtypeStruct((n, D), jnp.float32),
        mesh=mesh,
        scratch_types=[
            pltpu.VMEM((rows_per_w, CHUNK), jnp.int32),
            pltpu.VMEM((2, BURST, CHUNK, D), jnp.float32),
            pltpu.SemaphoreType.DMA,
            pltpu.SemaphoreType.DMA,
        ],
        compiler_params=pltpu.CompilerParams(use_tc_tiling_on_sc=False),
    )
    def gather_kernel(table_hbm, idx_hbm, out_hbm, idx_v, rows_v, gsem, osem):
        wid = lax.axis_index("s") * NC + lax.axis_index("c")
        w_base = wid * rows_per_w * CHUNK
        pltpu.sync_copy(idx_hbm.at[pl.ds(wid * rows_per_w, rows_per_w)], idx_v)

        def phase(p, s):
            @pl.when(p >= 2)
            def _():
                for t in range(BURST):
                    pltpu.make_async_copy(
                        rows_v.at[s, t], out_hbm.at[pl.ds(w_base, CHUNK)], osem
                    ).wait()

            descs = [
                pltpu.async_copy(
                    table_hbm.at[idx_v.at[p * BURST + t]], rows_v.at[s, t], gsem
                )
                for t in range(BURST)
            ]
            for d in descs:
                d.wait()
            for t in range(BURST):
                base = w_base + (p * BURST + t) * CHUNK
                pltpu.async_copy(rows_v.at[s, t], out_hbm.at[pl.ds(base, CHUNK)], osem)

        def outer(j, carry):
            phase(2 * j, 0)
            phase(2 * j + 1, 1)
            return carry

        lax.fori_loop(0, n_phases // 2, outer, 0)
        for _ in range(2 * BURST):
            pltpu.make_async_copy(
                rows_v.at[0, 0], out_hbm.at[pl.ds(w_base, CHUNK)], osem
            ).wait()

    return gather_kernel(table, idx2)


def _ln_body(emb_ref, pos_ref, gamma_ref, beta_ref, out_ref):
    h = emb_ref[...] * SQRT_D + pos_ref[...]
    mean = jnp.mean(h, axis=-1, keepdims=True)
    d = h - mean
    var = jnp.sum(d * d, axis=-1, keepdims=True) * (1.0 / (D - 1))
    out_ref[...] = gamma_ref[...] * (d / (jnp.sqrt(var) + EPS)) + beta_ref[...]


def _tc_layernorm(emb, pos, gamma, beta):
    b, s, d = emb.shape
    bb = 32
    grid = (b // bb,)
    return pl.pallas_call(
        _ln_body,
        grid=grid,
        in_specs=[
            pl.BlockSpec((bb, s, d), lambda i: (i, 0, 0)),
            pl.BlockSpec((1, s, d), lambda i: (0, 0, 0)),
            pl.BlockSpec((1, 1, d), lambda i: (0, 0, 0)),
            pl.BlockSpec((1, 1, d), lambda i: (0, 0, 0)),
        ],
        out_specs=pl.BlockSpec((bb, s, d), lambda i: (i, 0, 0)),
        out_shape=jax.ShapeDtypeStruct((b, s, d), jnp.float32),
    )(emb, pos, gamma, beta)


def kernel(x, table, gamma, beta, pos_embed):
    b, s = x.shape
    idx2 = x.astype(jnp.int32).reshape(b * s // CHUNK, CHUNK)
    gathered = _sc_gather(table, idx2)
    emb = gathered.reshape(b, s, D)
    pos = lax.slice(pos_embed, (0, 1, 0), (1, s + 1, D))
    return _tc_layernorm(emb, pos, gamma.reshape(1, 1, D), beta.reshape(1, 1, D))
